# baseline (speedup 1.0000x reference)
import functools
import math

import jax
import jax.numpy as jnp
from jax import lax
from jax.experimental import pallas as pl
from jax.experimental.pallas import tpu as pltpu
from jax.experimental.pallas import tpu_sc as plsc

D_MODEL = 64
SCALE = math.sqrt(D_MODEL)
CHUNK = 128
LANES = 16


@functools.cache
def _build(n_total: int, vocab: int):
    info = plsc.get_sparse_core_info()
    nw = info.num_cores * info.num_subcores
    per_w = n_total // nw
    n_chunks = per_w // CHUNK

    mesh = plsc.VectorSubcoreMesh(core_axis_name="c", subcore_axis_name="s")

    @functools.partial(
        pl.kernel,
        out_type=jax.ShapeDtypeStruct((n_total, D_MODEL), jnp.float32),
        mesh=mesh,
        scratch_types=[
            pltpu.VMEM((n_chunks, CHUNK), jnp.int32),
            pltpu.VMEM((CHUNK, D_MODEL), jnp.float32),
            pltpu.SemaphoreType.DMA,
        ],
        compiler_params=pltpu.CompilerParams(use_tc_tiling_on_sc=False),
    )
    def emb(x_hbm, table_hbm, out_hbm, idx_v, rows_v, sem):
        wid = lax.axis_index("s") * info.num_cores + lax.axis_index("c")
        base = wid * per_w
        pltpu.sync_copy(x_hbm.at[wid], idx_v)

        def chunk_body(j, carry):
            pltpu.async_copy(table_hbm.at[idx_v.at[j]], rows_v, sem).wait()

            def row_body(r, c2):
                for col in range(D_MODEL // LANES):
                    sl = pl.ds(col * LANES, LANES)
                    rows_v[r, sl] = rows_v[r, sl] * SCALE
                return c2

            lax.fori_loop(0, CHUNK, row_body, 0, unroll=2)
            pltpu.sync_copy(rows_v, out_hbm.at[pl.ds(base + j * CHUNK, CHUNK)])
            return carry

        lax.fori_loop(0, n_chunks, chunk_body, 0)

    return emb


def kernel(x, table):
    batch, hist = x.shape
    n_total = batch * hist
    emb = _build(n_total, table.shape[0])
    info = plsc.get_sparse_core_info()
    nw = info.num_cores * info.num_subcores
    per_w = n_total // nw
    xf = x.reshape(nw, per_w // CHUNK, CHUNK).astype(jnp.int32)
    out = emb(xf, table)
    return out.reshape(batch, hist, D_MODEL)


# speedup vs baseline: 1.0578x; 1.0578x over previous
import functools
import math

import jax
import jax.numpy as jnp
from jax import lax
from jax.experimental import pallas as pl
from jax.experimental.pallas import tpu as pltpu
from jax.experimental.pallas import tpu_sc as plsc

D_MODEL = 64
SCALE = math.sqrt(D_MODEL)
CHUNK = 128
LANES = 16
NBUF = 4


@functools.cache
def _build(n_total: int, vocab: int):
    info = plsc.get_sparse_core_info()
    nw = info.num_cores * info.num_subcores
    per_w = n_total // nw
    n_chunks = per_w // CHUNK
    n_groups = n_chunks // NBUF

    mesh = plsc.VectorSubcoreMesh(core_axis_name="c", subcore_axis_name="s")

    scratch = [
        pltpu.VMEM((n_chunks, CHUNK), jnp.int32),
        pltpu.VMEM((NBUF, CHUNK, D_MODEL), jnp.float32),
        pltpu.VMEM((NBUF, CHUNK, D_MODEL), jnp.float32),
    ]
    scratch += [pltpu.SemaphoreType.DMA] * (2 * NBUF)

    @functools.partial(
        pl.kernel,
        out_type=jax.ShapeDtypeStruct((n_total, D_MODEL), jnp.float32),
        mesh=mesh,
        scratch_types=scratch,
        compiler_params=pltpu.CompilerParams(use_tc_tiling_on_sc=False),
    )
    def emb(x_hbm, table_hbm, out_hbm, idx_v, g_v, s_v, *sems):
        gsem = sems[:NBUF]
        ssem = sems[NBUF:]
        wid = lax.axis_index("s") * info.num_cores + lax.axis_index("c")
        base = wid * per_w
        pltpu.sync_copy(x_hbm.at[wid], idx_v)

        for b in range(NBUF):
            pltpu.async_copy(table_hbm.at[idx_v.at[b]], g_v.at[b], gsem[b])

        def group_body(g, carry):
            for b in range(NBUF):
                jj = g * NBUF + b
                pltpu.make_async_copy(
                    table_hbm.at[idx_v.at[jj]], g_v.at[b], gsem[b]
                ).wait()

                @pl.when(jj >= NBUF)
                def _():
                    pltpu.make_async_copy(
                        s_v.at[b],
                        out_hbm.at[pl.ds(base + (jj - NBUF) * CHUNK, CHUNK)],
                        ssem[b],
                    ).wait()

                def row_body(r, c2):
                    for col in range(D_MODEL // LANES):
                        sl = pl.ds(col * LANES, LANES)
                        s_v[b, r, sl] = g_v[b, r, sl] * SCALE
                    return c2

                lax.fori_loop(0, CHUNK, row_body, 0, unroll=2)

                @pl.when(jj + NBUF < n_chunks)
                def _():
                    pltpu.async_copy(
                        table_hbm.at[idx_v.at[jj + NBUF]], g_v.at[b], gsem[b]
                    )

                pltpu.async_copy(
                    s_v.at[b],
                    out_hbm.at[pl.ds(base + jj * CHUNK, CHUNK)],
                    ssem[b],
                )
            return carry

        lax.fori_loop(0, n_groups, group_body, 0)

        for b in range(NBUF):
            jj = n_chunks - NBUF + b
            pltpu.make_async_copy(
                s_v.at[b],
                out_hbm.at[pl.ds(base + jj * CHUNK, CHUNK)],
                ssem[b],
            ).wait()

    return emb


def kernel(x, table):
    batch, hist = x.shape
    n_total = batch * hist
    emb = _build(n_total, table.shape[0])
    info = plsc.get_sparse_core_info()
    nw = info.num_cores * info.num_subcores
    per_w = n_total // nw
    xf = x.reshape(nw, per_w // CHUNK, CHUNK).astype(jnp.int32)
    out = emb(xf, table)
    return out.reshape(batch, hist, D_MODEL)
